# baseline (speedup 1.0000x reference)
import functools

import numpy as np
import jax
import jax.numpy as jnp
from jax import lax
from jax.experimental import pallas as pl
from jax.experimental.pallas import tpu as pltpu
from jax.experimental.pallas import tpu_sc as plsc

VOCAB_SIZE = 100000
D_MODEL = 1024
SEQ_LEN = 8192
BATCH = 4
N_ROWS = BATCH * SEQ_LEN

_NC = 2
_NS = 16
_NW = _NC * _NS
_S_PER_W = SEQ_LEN // _NW
_C = 4
_K = _S_PER_W // _C
_RPC = BATCH * _C
_NR = 6
_LANES = 16
_VECS_PER_ROW = D_MODEL // _LANES


def _sinusoid_pe_np(max_len: int, d_model: int) -> np.ndarray:
    pos = np.arange(max_len, dtype=np.float32)[:, None]
    i = np.arange(0, d_model, 2, dtype=np.float32)
    div = np.power(10000.0, i / d_model)
    pe = np.zeros((max_len, d_model), dtype=np.float32)
    pe[:, 0::2] = np.sin(pos / div)
    pe[:, 1::2] = np.cos(pos / div)
    return pe


_PE_NP = _sinusoid_pe_np(SEQ_LEN, D_MODEL)


def _make_kernel():
    mesh = plsc.VectorSubcoreMesh(core_axis_name="c", subcore_axis_name="s")

    ring_t = pltpu.VMEM((_RPC, D_MODEL), jnp.float32)
    pe_t = pltpu.VMEM((_C, D_MODEL), jnp.float32)
    dma = pltpu.SemaphoreType.DMA

    @functools.partial(
        pl.kernel,
        mesh=mesh,
        out_type=jax.ShapeDtypeStruct((N_ROWS, D_MODEL), jnp.float32),
        scratch_types=[
            pltpu.VMEM((_K, _RPC), jnp.int32),
            ring_t, ring_t, ring_t, ring_t, ring_t, ring_t,
            pe_t, pe_t, pe_t, pe_t, pe_t, pe_t,
            dma, dma, dma, dma, dma, dma,
            dma, dma, dma, dma, dma, dma,
            dma, dma, dma, dma, dma, dma,
        ],
    )
    def emb_kernel(xp_hbm, pe_hbm, table_hbm, out_hbm, idx_v,
                   r0, r1, r2, r3, r4, r5,
                   p0, p1, p2, p3, p4, p5,
                   g0, g1, g2, g3, g4, g5,
                   w0, w1, w2, w3, w4, w5,
                   s0, s1, s2, s3, s4, s5):
        wid = lax.axis_index("s") * _NC + lax.axis_index("c")
        s_base = wid * _S_PER_W

        rings = (r0, r1, r2, r3, r4, r5)
        pes = (p0, p1, p2, p3, p4, p5)
        gsems = (g0, g1, g2, g3, g4, g5)
        wsems = (w0, w1, w2, w3, w4, w5)
        psems = (s0, s1, s2, s3, s4, s5)

        def issue_gather(k, j):
            pltpu.async_copy(table_hbm.at[idx_v.at[k]], rings[j], gsems[j])

        def wait_gather(j):
            pltpu.make_async_copy(table_hbm.at[idx_v.at[0]],
                                  rings[j], gsems[j]).wait()

        def issue_pe(k, j):
            pltpu.async_copy(pe_hbm.at[pl.ds(s_base + k * _C, _C)],
                             pes[j], psems[j])

        def wait_pe(j):
            pltpu.make_async_copy(pe_hbm.at[pl.ds(0, _C)], pes[j],
                                  psems[j]).wait()

        def issue_writes(k, j):
            for b in range(BATCH):
                row0 = b * SEQ_LEN + s_base + k * _C
                pltpu.async_copy(rings[j].at[pl.ds(b * _C, _C)],
                                 out_hbm.at[pl.ds(row0, _C)], wsems[j])

        def drain_writes(j):
            for b in range(BATCH):
                pltpu.make_async_copy(rings[j].at[pl.ds(b * _C, _C)],
                                      out_hbm.at[pl.ds(0, _C)],
                                      wsems[j]).wait()

        def half(k, j, drain_prev=True, issue_next=True, prefetch_pe=True):
            nj = (j + 2) % _NR
            if drain_prev:
                drain_writes(nj)
            if issue_next:
                issue_gather(k + 2, nj)

            wait_pe(j)
            wait_gather(j)

            ring, pe_buf = rings[j], pes[j]

            @plsc.parallel_loop(0, _C * _VECS_PER_ROW, step=1, unroll=2)
            def _add_slice(t):
                r = lax.shift_right_logical(t, _VECS_PER_ROW.bit_length() - 1)
                v = lax.bitwise_and(t, _VECS_PER_ROW - 1)
                sl = pl.ds(v * _LANES, _LANES)
                val = pe_buf[r, sl]
                for b in range(BATCH):
                    plsc.addupdate(ring.at[b * _C + r, sl], val)

            if prefetch_pe:
                issue_pe(k + 4, (j + 4) % _NR)
            issue_writes(k, j)

        pltpu.sync_copy(xp_hbm.at[wid], idx_v)
        issue_pe(0, 0)
        issue_pe(1, 1)
        issue_pe(2, 2)
        issue_pe(3, 3)
        issue_gather(0, 0)
        issue_gather(1, 1)

        half(0, 0, drain_prev=False)
        half(1, 1, drain_prev=False)
        half(2, 2, drain_prev=False)
        half(3, 3, drain_prev=False)

        def body(m, carry):
            k = 4 + 6 * m
            for i in range(6):
                half(k + i, (4 + i) % _NR)
            return carry

        lax.fori_loop(0, (_K - 10) // 6, body, 0, unroll=False)

        half(_K - 6, 4)
        half(_K - 5, 5)
        half(_K - 4, 0, prefetch_pe=False)
        half(_K - 3, 1, prefetch_pe=False)
        half(_K - 2, 2, issue_next=False, prefetch_pe=False)
        half(_K - 1, 3, issue_next=False, prefetch_pe=False)
        drain_writes(0)
        drain_writes(1)
        drain_writes(2)
        drain_writes(3)

    return emb_kernel


_EMB_KERNEL = _make_kernel()


def kernel(x, table):
    pe = jnp.asarray(_PE_NP)
    xp = (x.astype(jnp.int32)
          .reshape(BATCH, _NW, _K, _C)
          .transpose(1, 2, 0, 3)
          .reshape(_NW, _K, _RPC))
    out = _EMB_KERNEL(xp, pe, table)
    return out.reshape(BATCH, SEQ_LEN, D_MODEL)


# speedup vs baseline: 1.0271x; 1.0271x over previous
import functools

import numpy as np
import jax
import jax.numpy as jnp
from jax import lax
from jax.experimental import pallas as pl
from jax.experimental.pallas import tpu as pltpu
from jax.experimental.pallas import tpu_sc as plsc

VOCAB_SIZE = 100000
D_MODEL = 1024
SEQ_LEN = 8192
BATCH = 4
N_ROWS = BATCH * SEQ_LEN

_NC = 2
_NS = 16
_NW = _NC * _NS
_S_PER_W = SEQ_LEN // _NW
_C = 8
_K = _S_PER_W // _C
_RPC = BATCH * _C
_LANES = 16
_VECS_PER_ROW = D_MODEL // _LANES


def _sinusoid_pe_np(max_len: int, d_model: int) -> np.ndarray:
    pos = np.arange(max_len, dtype=np.float32)[:, None]
    i = np.arange(0, d_model, 2, dtype=np.float32)
    div = np.power(10000.0, i / d_model)
    pe = np.zeros((max_len, d_model), dtype=np.float32)
    pe[:, 0::2] = np.sin(pos / div)
    pe[:, 1::2] = np.cos(pos / div)
    return pe


_PE_NP = _sinusoid_pe_np(SEQ_LEN, D_MODEL)


def _make_kernel():
    mesh = plsc.VectorSubcoreMesh(core_axis_name="c", subcore_axis_name="s")

    ring_t = pltpu.VMEM((_RPC, D_MODEL), jnp.float32)
    pe_t = pltpu.VMEM((_C, D_MODEL), jnp.float32)
    dma = pltpu.SemaphoreType.DMA

    @functools.partial(
        pl.kernel,
        mesh=mesh,
        out_type=jax.ShapeDtypeStruct((N_ROWS, D_MODEL), jnp.float32),
        scratch_types=[
            pltpu.VMEM((_K, _RPC), jnp.int32),
            ring_t, ring_t, ring_t,
            pe_t, pe_t, pe_t,
            dma, dma, dma,
            dma, dma, dma,
            dma, dma, dma,
        ],
    )
    def emb_kernel(xp_hbm, pe_hbm, table_hbm, out_hbm, idx_v,
                   r0, r1, r2, p0, p1, p2,
                   g0, g1, g2, w0, w1, w2, s0, s1, s2):
        wid = lax.axis_index("s") * _NC + lax.axis_index("c")
        s_base = wid * _S_PER_W

        rings = (r0, r1, r2)
        pes = (p0, p1, p2)
        gsems = (g0, g1, g2)
        wsems = (w0, w1, w2)
        psems = (s0, s1, s2)

        def issue_gather(k, j):
            pltpu.async_copy(table_hbm.at[idx_v.at[k]], rings[j], gsems[j])

        def wait_gather(j):
            pltpu.make_async_copy(table_hbm.at[idx_v.at[0]],
                                  rings[j], gsems[j]).wait()

        def issue_pe(k, j):
            pltpu.async_copy(pe_hbm.at[pl.ds(s_base + k * _C, _C)],
                             pes[j], psems[j])

        def wait_pe(j):
            pltpu.make_async_copy(pe_hbm.at[pl.ds(0, _C)], pes[j],
                                  psems[j]).wait()

        def issue_writes(k, j):
            for b in range(BATCH):
                row0 = b * SEQ_LEN + s_base + k * _C
                pltpu.async_copy(rings[j].at[pl.ds(b * _C, _C)],
                                 out_hbm.at[pl.ds(row0, _C)], wsems[j])

        def drain_writes(j):
            for b in range(BATCH):
                pltpu.make_async_copy(rings[j].at[pl.ds(b * _C, _C)],
                                      out_hbm.at[pl.ds(0, _C)],
                                      wsems[j]).wait()

        def half(k, j, drain_prev=True, issue_next=True, prefetch_pe=True):
            nj = (j + 1) % 3
            if drain_prev:
                drain_writes(nj)
            if issue_next:
                issue_gather(k + 1, nj)

            wait_pe(j)
            wait_gather(j)

            ring, pe_buf = rings[j], pes[j]

            @plsc.parallel_loop(0, _C * _VECS_PER_ROW, step=1, unroll=4)
            def _add_slice(t):
                r = lax.shift_right_logical(t, _VECS_PER_ROW.bit_length() - 1)
                v = lax.bitwise_and(t, _VECS_PER_ROW - 1)
                sl = pl.ds(v * _LANES, _LANES)
                val = pe_buf[r, sl]
                for b in range(BATCH):
                    plsc.addupdate(ring.at[b * _C + r, sl], val)

            if prefetch_pe:
                issue_pe(k + 2, (j + 2) % 3)
            issue_writes(k, j)

        pltpu.sync_copy(xp_hbm.at[wid], idx_v)
        issue_pe(0, 0)
        issue_pe(1, 1)
        issue_gather(0, 0)

        half(0, 0, drain_prev=False)
        half(1, 1, drain_prev=False)

        def body(kk, carry):
            k = 2 + 3 * kk
            half(k, 2)
            half(k + 1, 0)
            half(k + 2, 1)
            return carry

        lax.fori_loop(0, (_K - 5) // 3, body, 0, unroll=False)

        half(_K - 3, 2)
        half(_K - 2, 0, prefetch_pe=False)
        half(_K - 1, 1, issue_next=False, prefetch_pe=False)
        drain_writes(0)
        drain_writes(1)

    return emb_kernel


_EMB_KERNEL = _make_kernel()


def kernel(x, table):
    pe = jnp.asarray(_PE_NP)
    xp = (x.astype(jnp.int32)
          .reshape(BATCH, _NW, _K, _C)
          .transpose(1, 2, 0, 3)
          .reshape(_NW, _K, _RPC))
    out = _EMB_KERNEL(xp, pe, table)
    return out.reshape(BATCH, SEQ_LEN, D_MODEL)


# speedup vs baseline: 1.0378x; 1.0104x over previous
import functools

import numpy as np
import jax
import jax.numpy as jnp
from jax import lax
from jax.experimental import pallas as pl
from jax.experimental.pallas import tpu as pltpu
from jax.experimental.pallas import tpu_sc as plsc

VOCAB_SIZE = 100000
D_MODEL = 1024
SEQ_LEN = 8192
BATCH = 4
N_ROWS = BATCH * SEQ_LEN

_NC = 2
_NS = 16
_NW = _NC * _NS
_S_PER_W = SEQ_LEN // _NW
_C = 8
_K = _S_PER_W // _C
_LANES = 16
_VECS_PER_ROW = D_MODEL // _LANES


def _sinusoid_pe_np(max_len: int, d_model: int) -> np.ndarray:
    pos = np.arange(max_len, dtype=np.float32)[:, None]
    i = np.arange(0, d_model, 2, dtype=np.float32)
    div = np.power(10000.0, i / d_model)
    pe = np.zeros((max_len, d_model), dtype=np.float32)
    pe[:, 0::2] = np.sin(pos / div)
    pe[:, 1::2] = np.cos(pos / div)
    return pe


_PE_NP = _sinusoid_pe_np(SEQ_LEN, D_MODEL)


def _make_kernel():
    mesh = plsc.VectorSubcoreMesh(core_axis_name="c", subcore_axis_name="s")

    ring_t = pltpu.VMEM((BATCH, _C, D_MODEL), jnp.float32)
    pe_t = pltpu.VMEM((_C, D_MODEL), jnp.float32)
    dma = pltpu.SemaphoreType.DMA

    @functools.partial(
        pl.kernel,
        mesh=mesh,
        out_type=jax.ShapeDtypeStruct((N_ROWS, D_MODEL), jnp.float32),
        scratch_types=[
            pltpu.VMEM((BATCH, _S_PER_W), jnp.int32),
            ring_t, ring_t, ring_t,
            pe_t, pe_t, pe_t,
            dma, dma, dma,
            dma, dma, dma,
            dma, dma, dma,
        ],
    )
    def emb_kernel(x_hbm, pe_hbm, table_hbm, out_hbm,
                   idx_v, r0, r1, r2, p0, p1, p2,
                   g0, g1, g2, w0, w1, w2, s0, s1, s2):
        wid = lax.axis_index("s") * _NC + lax.axis_index("c")
        s_base = wid * _S_PER_W

        rings = (r0, r1, r2)
        pes = (p0, p1, p2)
        gsems = (g0, g1, g2)
        wsems = (w0, w1, w2)
        psems = (s0, s1, s2)

        def issue_gathers(k, j):
            for b in range(BATCH):
                pltpu.async_copy(
                    table_hbm.at[idx_v.at[b, pl.ds(k * _C, _C)]],
                    rings[j].at[b], gsems[j])

        def issue_pe(k, j):
            pltpu.async_copy(pe_hbm.at[pl.ds(s_base + k * _C, _C)],
                             pes[j], psems[j])

        def wait_pe(j):
            pltpu.make_async_copy(pe_hbm.at[pl.ds(0, _C)], pes[j],
                                  psems[j]).wait()

        def drain_writes(j):
            for b in range(BATCH):
                pltpu.make_async_copy(rings[j].at[b],
                                      out_hbm.at[pl.ds(0, _C)],
                                      wsems[j]).wait()

        def drain_gathers(j):
            for b in range(BATCH):
                pltpu.make_async_copy(
                    table_hbm.at[idx_v.at[b, pl.ds(0, _C)]],
                    rings[j].at[b], gsems[j]).wait()

        def issue_writes(k, j):
            for b in range(BATCH):
                row0 = b * SEQ_LEN + s_base + k * _C
                pltpu.async_copy(rings[j].at[b],
                                 out_hbm.at[pl.ds(row0, _C)], wsems[j])

        def half(k, j, drain_prev=True, issue_next=True, prefetch_pe=True):
            nj = (j + 1) % 3
            if drain_prev:
                drain_writes(nj)
            if issue_next:
                issue_gathers(k + 1, nj)

            wait_pe(j)
            drain_gathers(j)

            ring, pe_buf = rings[j], pes[j]

            @plsc.parallel_loop(0, _C * _VECS_PER_ROW, step=1, unroll=4)
            def _add_slice(t):
                r = lax.shift_right_logical(t, _VECS_PER_ROW.bit_length() - 1)
                v = lax.bitwise_and(t, _VECS_PER_ROW - 1)
                sl = pl.ds(v * _LANES, _LANES)
                val = pe_buf[r, sl]
                for b in range(BATCH):
                    plsc.addupdate(ring.at[b, r, sl], val)

            if prefetch_pe:
                issue_pe(k + 2, (j + 2) % 3)
            issue_writes(k, j)

        for b in range(BATCH):
            pltpu.sync_copy(x_hbm.at[pl.ds(b * SEQ_LEN + s_base, _S_PER_W)],
                            idx_v.at[b])
        issue_pe(0, 0)
        issue_pe(1, 1)
        issue_gathers(0, 0)

        half(0, 0, drain_prev=False)
        half(1, 1, drain_prev=False)

        def body(kk, carry):
            k = 2 + 3 * kk
            half(k, 2)
            half(k + 1, 0)
            half(k + 2, 1)
            return carry

        lax.fori_loop(0, (_K - 5) // 3, body, 0, unroll=False)

        half(_K - 3, 2)
        half(_K - 2, 0, prefetch_pe=False)
        half(_K - 1, 1, issue_next=False, prefetch_pe=False)
        drain_writes(0)
        drain_writes(1)

    return emb_kernel


_EMB_KERNEL = _make_kernel()


def kernel(x, table):
    pe = jnp.asarray(_PE_NP)
    xflat = x.reshape(N_ROWS).astype(jnp.int32)
    out = _EMB_KERNEL(xflat, pe, table)
    return out.reshape(BATCH, SEQ_LEN, D_MODEL)
